# baseline (speedup 1.0000x reference)
import jax
import jax.numpy as jnp
from jax import lax
from jax.experimental import pallas as pl
from jax.experimental.pallas import tpu as pltpu
from jax.experimental.pallas import tpu_sc as plsc

KMER_SIZE = 4
BATCH = 1024
SEQ_LEN = 512
EMBED_DIM = 64

NUM_CORES = 2
NUM_SUBCORES = 16
NUM_WORKERS = NUM_CORES * NUM_SUBCORES
PER_WORKER = BATCH * SEQ_LEN // NUM_WORKERS
CHUNK = 256
N_CHUNKS = PER_WORKER // CHUNK
PROWS = CHUNK // 2
OUT_ROWS = BATCH * SEQ_LEN * EMBED_DIM // 128
CHUNKS_PER_SEQ = SEQ_LEN // CHUNK


def _sc_body(idx_hbm, table_hbm, out_hbm, idx_v, g_v, p_v, g0, g1, s0, s1):
    gsems = (g0, g1)
    ssems = (s0, s1)
    wid = lax.axis_index("s") * NUM_CORES + lax.axis_index("c")
    out_base = wid * N_CHUNKS * PROWS
    pltpu.sync_copy(idx_hbm.at[pl.ds(wid * N_CHUNKS, N_CHUNKS)], idx_v)

    def fire_gather(c, slot):
        pltpu.async_copy(table_hbm.at[idx_v.at[c]], g_v.at[slot], gsems[slot])

    def wait_gather(slot):
        pltpu.make_async_copy(
            table_hbm.at[idx_v.at[0]], g_v.at[slot], gsems[slot]
        ).wait()

    def fire_scatter(c, slot):
        pltpu.async_copy(
            p_v.at[slot], out_hbm.at[pl.ds(out_base + c * PROWS, PROWS)],
            ssems[slot],
        )

    def wait_scatter(slot):
        pltpu.make_async_copy(
            p_v.at[slot], out_hbm.at[pl.ds(0, PROWS)], ssems[slot]
        ).wait()

    def repack(slot):
        def rows4(i, _):
            r0 = i * 4
            for dr in range(4):
                r = r0 + dr
                for l in range(EMBED_DIM // 16):
                    p_v[slot, r, pl.ds(l * 16, 16)] = g_v[
                        slot, r, pl.ds(l * 16, 16)
                    ]
                    p_v[slot, r, pl.ds(EMBED_DIM + l * 16, 16)] = g_v[
                        slot, PROWS + r, pl.ds(l * 16, 16)
                    ]
            return 0

        lax.fori_loop(0, PROWS // 4, rows4, 0)

    def mask(slot):
        zeros = jnp.zeros((16,), jnp.float32)
        for r in range(KMER_SIZE - 1):
            for l in range(EMBED_DIM // 16):
                p_v[slot, r, pl.ds(l * 16, 16)] = zeros

    fire_gather(0, 0)
    fire_gather(1, 1)

    wait_gather(0)
    repack(0)
    mask(0)
    fire_scatter(0, 0)
    fire_gather(2, 0)

    wait_gather(1)
    repack(1)
    fire_scatter(1, 1)
    fire_gather(3, 1)

    def group(g, _):
        for b in range(2):
            c = 2 * g + 2 + b
            wait_gather(b)
            wait_scatter(b)
            repack(b)
            if b == 0:
                mask(b)
            fire_scatter(c, b)

            @pl.when(c + 2 < N_CHUNKS)
            def _():
                fire_gather(c + 2, b)

        return 0

    lax.fori_loop(0, (N_CHUNKS - 2) // 2, group, 0)

    wait_scatter(0)
    wait_scatter(1)


def _tc_unpack_body(p_ref, out_ref):
    x = p_ref[0, 0]
    out_ref[0, 0:PROWS, :] = x[:, 0:EMBED_DIM]
    out_ref[0, PROWS:CHUNK, :] = x[:, EMBED_DIM:128]


@jax.jit
def _encode(kmer_indices, kmer_table):
    idx2d = kmer_indices.reshape(NUM_WORKERS * N_CHUNKS, CHUNK)
    mesh = plsc.VectorSubcoreMesh(
        core_axis_name="c",
        subcore_axis_name="s",
        num_cores=NUM_CORES,
        num_subcores=NUM_SUBCORES,
    )
    run = pl.kernel(
        _sc_body,
        out_type=jax.ShapeDtypeStruct((OUT_ROWS, 128), jnp.float32),
        mesh=mesh,
        scratch_types=[
            pltpu.VMEM((N_CHUNKS, CHUNK), jnp.int32),
            pltpu.VMEM((2, CHUNK, EMBED_DIM), jnp.float32),
            pltpu.VMEM((2, PROWS, 128), jnp.float32),
            pltpu.SemaphoreType.DMA,
            pltpu.SemaphoreType.DMA,
            pltpu.SemaphoreType.DMA,
            pltpu.SemaphoreType.DMA,
        ],
        compiler_params=pltpu.CompilerParams(use_tc_tiling_on_sc=False),
    )
    packed = run(idx2d, kmer_table)
    packed4 = packed.reshape(BATCH, CHUNKS_PER_SEQ, PROWS, 128)
    out = pl.pallas_call(
        _tc_unpack_body,
        grid=(BATCH, CHUNKS_PER_SEQ),
        in_specs=[
            pl.BlockSpec((1, 1, PROWS, 128), lambda b, h: (b, h, 0, 0)),
        ],
        out_specs=pl.BlockSpec((1, CHUNK, EMBED_DIM), lambda b, h: (b, h, 0)),
        out_shape=jax.ShapeDtypeStruct((BATCH, SEQ_LEN, EMBED_DIM), jnp.float32),
    )(packed4)
    return out


def kernel(kmer_indices, kmer_table):
    return _encode(kmer_indices, kmer_table)


# speedup vs baseline: 1.9505x; 1.9505x over previous
import jax
import jax.numpy as jnp
from jax import lax
from jax.experimental import pallas as pl
from jax.experimental.pallas import tpu as pltpu
from jax.experimental.pallas import tpu_sc as plsc

KMER_SIZE = 4
BATCH = 1024
SEQ_LEN = 512
EMBED_DIM = 64

NUM_CORES = 2
NUM_SUBCORES = 16
NUM_WORKERS = NUM_CORES * NUM_SUBCORES
PER_WORKER = BATCH * SEQ_LEN // NUM_WORKERS
CHUNK = 256
N_CHUNKS = PER_WORKER // CHUNK
PROWS = CHUNK // 2
OUT_ROWS = BATCH * SEQ_LEN * EMBED_DIM // 128
CHUNKS_PER_SEQ = SEQ_LEN // CHUNK


def _sc_body(idx_hbm, table_hbm, out_hbm, idx_v, g_v, p_v, g0, g1, s0, s1):
    gsems = (g0, g1)
    ssems = (s0, s1)
    wid = lax.axis_index("s") * NUM_CORES + lax.axis_index("c")
    out_base = wid * N_CHUNKS * PROWS
    pltpu.sync_copy(idx_hbm.at[pl.ds(wid * N_CHUNKS, N_CHUNKS)], idx_v)

    def fire_gather(c, slot):
        pltpu.async_copy(table_hbm.at[idx_v.at[c]], g_v.at[slot], gsems[slot])

    def wait_gather(slot):
        pltpu.make_async_copy(
            table_hbm.at[idx_v.at[0]], g_v.at[slot], gsems[slot]
        ).wait()

    def fire_scatter(c, slot):
        pltpu.async_copy(
            p_v.at[slot], out_hbm.at[pl.ds(out_base + c * PROWS, PROWS)],
            ssems[slot],
        )

    def wait_scatter(slot):
        pltpu.make_async_copy(
            p_v.at[slot], out_hbm.at[pl.ds(0, PROWS)], ssems[slot]
        ).wait()

    def repack(slot):
        def rows4(i, _):
            r0 = i * 4
            for dr in range(4):
                r = r0 + dr
                for l in range(EMBED_DIM // 16):
                    p_v[slot, r, pl.ds(l * 16, 16)] = g_v[
                        slot, r, pl.ds(l * 16, 16)
                    ]
                    p_v[slot, r, pl.ds(EMBED_DIM + l * 16, 16)] = g_v[
                        slot, PROWS + r, pl.ds(l * 16, 16)
                    ]
            return 0

        lax.fori_loop(0, PROWS // 4, rows4, 0)

    def mask(slot):
        zeros = jnp.zeros((16,), jnp.float32)
        for r in range(KMER_SIZE - 1):
            for l in range(EMBED_DIM // 16):
                p_v[slot, r, pl.ds(l * 16, 16)] = zeros

    fire_gather(0, 0)
    fire_gather(1, 1)

    wait_gather(0)
    repack(0)
    mask(0)
    fire_scatter(0, 0)
    fire_gather(2, 0)

    wait_gather(1)
    repack(1)
    fire_scatter(1, 1)
    fire_gather(3, 1)

    def group(g, _):
        for b in range(2):
            c = 2 * g + 2 + b
            wait_gather(b)
            wait_scatter(b)
            repack(b)
            if b == 0:
                mask(b)
            fire_scatter(c, b)

            @pl.when(c + 2 < N_CHUNKS)
            def _():
                fire_gather(c + 2, b)

        return 0

    lax.fori_loop(0, (N_CHUNKS - 2) // 2, group, 0)

    wait_scatter(0)
    wait_scatter(1)


def _tc_unpack_body(p_ref, out_ref):
    x = p_ref[0, 0]
    out_ref[0, 0:PROWS, :] = x[:, 0:EMBED_DIM]
    out_ref[0, PROWS:CHUNK, :] = x[:, EMBED_DIM:128]


@jax.jit
def _encode(kmer_indices, kmer_table):
    idx2d = kmer_indices.reshape(NUM_WORKERS * N_CHUNKS, CHUNK)
    mesh = plsc.VectorSubcoreMesh(
        core_axis_name="c",
        subcore_axis_name="s",
        num_cores=NUM_CORES,
        num_subcores=NUM_SUBCORES,
    )
    run = pl.kernel(
        _sc_body,
        out_type=jax.ShapeDtypeStruct((OUT_ROWS, 128), jnp.float32),
        mesh=mesh,
        scratch_types=[
            pltpu.VMEM((N_CHUNKS, CHUNK), jnp.int32),
            pltpu.VMEM((2, CHUNK, EMBED_DIM), jnp.float32),
            pltpu.VMEM((2, PROWS, 128), jnp.float32),
            pltpu.SemaphoreType.DMA,
            pltpu.SemaphoreType.DMA,
            pltpu.SemaphoreType.DMA,
            pltpu.SemaphoreType.DMA,
        ],
        compiler_params=pltpu.CompilerParams(use_tc_tiling_on_sc=False),
    )
    packed = run(idx2d, kmer_table)
    x5 = packed.reshape(BATCH, CHUNKS_PER_SEQ, PROWS, 2, EMBED_DIM)
    out = x5.transpose(0, 1, 3, 2, 4).reshape(BATCH, SEQ_LEN, EMBED_DIM)
    return out


def kernel(kmer_indices, kmer_table):
    return _encode(kmer_indices, kmer_table)


# speedup vs baseline: 2.5169x; 1.2904x over previous
import jax
import jax.numpy as jnp
from jax import lax
from jax.experimental import pallas as pl
from jax.experimental.pallas import tpu as pltpu
from jax.experimental.pallas import tpu_sc as plsc

KMER_SIZE = 4
BATCH = 1024
SEQ_LEN = 512
EMBED_DIM = 64
NSPLIT = 2
HBATCH = BATCH // NSPLIT

NUM_CORES = 2
NUM_SUBCORES = 16
NUM_WORKERS = NUM_CORES * NUM_SUBCORES
SEQS_PER_WORKER = HBATCH // NUM_WORKERS
CHUNK = SEQ_LEN
N_CHUNKS = SEQS_PER_WORKER


def _sc_body(idx_hbm, table_hbm, out_hbm, idx_v, rows_v, g0, g1, s0, s1):
    gsems = (g0, g1)
    ssems = (s0, s1)
    wid = lax.axis_index("s") * NUM_CORES + lax.axis_index("c")
    seq_base = wid * SEQS_PER_WORKER
    pltpu.sync_copy(idx_hbm.at[pl.ds(seq_base, N_CHUNKS)], idx_v)

    def fire_gather(c, slot):
        pltpu.async_copy(table_hbm.at[idx_v.at[c]], rows_v.at[slot], gsems[slot])

    def wait_gather(slot):
        pltpu.make_async_copy(
            table_hbm.at[idx_v.at[0]], rows_v.at[slot], gsems[slot]
        ).wait()

    def fire_scatter(c, slot):
        pltpu.async_copy(rows_v.at[slot], out_hbm.at[seq_base + c], ssems[slot])

    def wait_scatter(slot):
        pltpu.make_async_copy(
            rows_v.at[slot], out_hbm.at[seq_base], ssems[slot]
        ).wait()

    def mask(slot):
        zeros = jnp.zeros((16,), jnp.float32)
        for r in range(KMER_SIZE - 1):
            for l in range(EMBED_DIM // 16):
                rows_v[slot, r, pl.ds(l * 16, 16)] = zeros

    fire_gather(0, 0)
    wait_gather(0)
    mask(0)
    fire_scatter(0, 0)
    fire_gather(1, 1)

    def group(g, _):
        for b in range(2):
            c = 2 * g + 1 + b
            slot = (1 + b) % 2
            wait_gather(slot)
            mask(slot)
            fire_scatter(c, slot)
            wait_scatter(1 - slot)
            fire_gather(c + 1, 1 - slot)
        return 0

    lax.fori_loop(0, (N_CHUNKS - 2) // 2, group, 0)

    wait_gather(1)
    mask(1)
    fire_scatter(N_CHUNKS - 1, 1)
    wait_scatter(0)
    wait_scatter(1)


@jax.jit
def _encode(kmer_indices, kmer_table):
    mesh = plsc.VectorSubcoreMesh(
        core_axis_name="c",
        subcore_axis_name="s",
        num_cores=NUM_CORES,
        num_subcores=NUM_SUBCORES,
    )
    run = pl.kernel(
        _sc_body,
        out_type=jax.ShapeDtypeStruct((HBATCH, SEQ_LEN, EMBED_DIM), jnp.float32),
        mesh=mesh,
        scratch_types=[
            pltpu.VMEM((N_CHUNKS, CHUNK), jnp.int32),
            pltpu.VMEM((2, CHUNK, EMBED_DIM), jnp.float32),
            pltpu.SemaphoreType.DMA,
            pltpu.SemaphoreType.DMA,
            pltpu.SemaphoreType.DMA,
            pltpu.SemaphoreType.DMA,
        ],
        compiler_params=pltpu.CompilerParams(use_tc_tiling_on_sc=False),
    )
    halves = [
        run(lax.slice_in_dim(kmer_indices, h * HBATCH, (h + 1) * HBATCH, axis=0),
            kmer_table)
        for h in range(NSPLIT)
    ]
    return jnp.concatenate(halves, axis=0)


def kernel(kmer_indices, kmer_table):
    return _encode(kmer_indices, kmer_table)


# speedup vs baseline: 3.2934x; 1.3085x over previous
import jax
import jax.numpy as jnp
from jax import lax
from jax.experimental import pallas as pl
from jax.experimental.pallas import tpu as pltpu
from jax.experimental.pallas import tpu_sc as plsc

KMER_SIZE = 4
BATCH = 1024
SEQ_LEN = 512
EMBED_DIM = 64

NUM_CORES = 2
NUM_SUBCORES = 16
NUM_WORKERS = NUM_CORES * NUM_SUBCORES
SEQS_PER_WORKER = BATCH // NUM_WORKERS
PER_WORKER = SEQS_PER_WORKER * SEQ_LEN
CHUNK = SEQ_LEN
N_CHUNKS = PER_WORKER // CHUNK


def _sc_body(idx_hbm, table_hbm, out_hbm, idx_v, rows_v, g0, g1, s0, s1):
    gsems = (g0, g1)
    ssems = (s0, s1)
    wid = lax.axis_index("s") * NUM_CORES + lax.axis_index("c")
    seq_base = wid * SEQS_PER_WORKER
    pltpu.sync_copy(idx_hbm.at[pl.ds(seq_base, N_CHUNKS)], idx_v)

    def fire_gather(c, slot):
        pltpu.async_copy(table_hbm.at[idx_v.at[c]], rows_v.at[slot], gsems[slot])

    def wait_gather(slot):
        pltpu.make_async_copy(
            table_hbm.at[idx_v.at[0]], rows_v.at[slot], gsems[slot]
        ).wait()

    def fire_scatter(c, slot):
        pltpu.async_copy(rows_v.at[slot], out_hbm.at[seq_base + c], ssems[slot])

    def wait_scatter(slot):
        pltpu.make_async_copy(
            rows_v.at[slot], out_hbm.at[seq_base], ssems[slot]
        ).wait()

    def mask(slot):
        zeros = jnp.zeros((16,), jnp.float32)
        for r in range(KMER_SIZE - 1):
            for l in range(EMBED_DIM // 16):
                rows_v[slot, r, pl.ds(l * 16, 16)] = zeros

    fire_gather(0, 0)
    wait_gather(0)
    mask(0)
    fire_scatter(0, 0)
    fire_gather(1, 1)

    def group(g, _):
        for b in range(2):
            c = 2 * g + 1 + b
            slot = (1 + b) % 2
            wait_gather(slot)
            mask(slot)
            fire_scatter(c, slot)
            wait_scatter(1 - slot)
            fire_gather(c + 1, 1 - slot)
        return 0

    lax.fori_loop(0, (N_CHUNKS - 2) // 2, group, 0)

    wait_gather(1)
    mask(1)
    fire_scatter(N_CHUNKS - 1, 1)
    wait_scatter(0)
    wait_scatter(1)


@jax.jit
def _encode(kmer_indices, kmer_table):
    mesh = plsc.VectorSubcoreMesh(
        core_axis_name="c",
        subcore_axis_name="s",
        num_cores=NUM_CORES,
        num_subcores=NUM_SUBCORES,
    )
    run = pl.kernel(
        _sc_body,
        out_type=jax.ShapeDtypeStruct((BATCH, SEQ_LEN, EMBED_DIM), jnp.float32),
        mesh=mesh,
        scratch_types=[
            pltpu.VMEM((N_CHUNKS, CHUNK), jnp.int32),
            pltpu.VMEM((2, CHUNK, EMBED_DIM), jnp.float32),
            pltpu.SemaphoreType.DMA,
            pltpu.SemaphoreType.DMA,
            pltpu.SemaphoreType.DMA,
            pltpu.SemaphoreType.DMA,
        ],
        compiler_params=pltpu.CompilerParams(
            use_tc_tiling_on_sc=False, needs_layout_passes=False
        ),
    )
    return run(kmer_indices, kmer_table)


def kernel(kmer_indices, kmer_table):
    return _encode(kmer_indices, kmer_table)


# speedup vs baseline: 3.3191x; 1.0078x over previous
import jax
import jax.numpy as jnp
from jax import lax
from jax.experimental import pallas as pl
from jax.experimental.pallas import tpu as pltpu
from jax.experimental.pallas import tpu_sc as plsc

KMER_SIZE = 4
BATCH = 1024
SEQ_LEN = 512
EMBED_DIM = 64

NUM_CORES = 2
NUM_SUBCORES = 16
NUM_WORKERS = NUM_CORES * NUM_SUBCORES
SEQS_PER_WORKER = BATCH // NUM_WORKERS
PER_WORKER = SEQS_PER_WORKER * SEQ_LEN
CHUNK = SEQ_LEN
N_CHUNKS = PER_WORKER // CHUNK


def _sc_body(idx_hbm, table_hbm, out_hbm, idx_v, rows_v,
             g0, g1, g2, s0, s1, s2):
    gsems = (g0, g1, g2)
    ssems = (s0, s1, s2)
    wid = lax.axis_index("s") * NUM_CORES + lax.axis_index("c")
    seq_base = wid * SEQS_PER_WORKER
    pltpu.sync_copy(idx_hbm.at[pl.ds(seq_base, N_CHUNKS)], idx_v)

    def fire_gather(c, slot):
        pltpu.async_copy(table_hbm.at[idx_v.at[c]], rows_v.at[slot], gsems[slot])

    def wait_gather(slot):
        pltpu.make_async_copy(
            table_hbm.at[idx_v.at[0]], rows_v.at[slot], gsems[slot]
        ).wait()

    def fire_scatter(c, slot):
        pltpu.async_copy(rows_v.at[slot], out_hbm.at[seq_base + c], ssems[slot])

    def wait_scatter(slot):
        pltpu.make_async_copy(
            rows_v.at[slot], out_hbm.at[seq_base], ssems[slot]
        ).wait()

    def mask(slot):
        zeros = jnp.zeros((16,), jnp.float32)
        for r in range(KMER_SIZE - 1):
            for l in range(EMBED_DIM // 16):
                rows_v[slot, r, pl.ds(l * 16, 16)] = zeros

    fire_gather(0, 0)
    fire_gather(1, 1)

    wait_gather(0)
    mask(0)
    fire_scatter(0, 0)
    fire_gather(2, 2)

    wait_gather(1)
    mask(1)
    fire_scatter(1, 1)
    wait_scatter(0)
    fire_gather(3, 0)

    def group(g, _):
        for b in range(3):
            c = 3 * g + 2 + b
            slot = (2 + b) % 3
            wait_gather(slot)
            mask(slot)
            fire_scatter(c, slot)

            @pl.when(c + 2 < N_CHUNKS)
            def _():
                wait_scatter((slot + 2) % 3)
                fire_gather(c + 2, (slot + 2) % 3)

        return 0

    lax.fori_loop(0, (N_CHUNKS - 2) // 3, group, 0)

    wait_scatter(2)
    wait_scatter(0)
    wait_scatter(1)


@jax.jit
def _encode(kmer_indices, kmer_table):
    mesh = plsc.VectorSubcoreMesh(
        core_axis_name="c",
        subcore_axis_name="s",
        num_cores=NUM_CORES,
        num_subcores=NUM_SUBCORES,
    )
    run = pl.kernel(
        _sc_body,
        out_type=jax.ShapeDtypeStruct((BATCH, SEQ_LEN, EMBED_DIM), jnp.float32),
        mesh=mesh,
        scratch_types=[
            pltpu.VMEM((N_CHUNKS, CHUNK), jnp.int32),
            pltpu.VMEM((3, CHUNK, EMBED_DIM), jnp.float32),
            pltpu.SemaphoreType.DMA,
            pltpu.SemaphoreType.DMA,
            pltpu.SemaphoreType.DMA,
            pltpu.SemaphoreType.DMA,
            pltpu.SemaphoreType.DMA,
            pltpu.SemaphoreType.DMA,
        ],
        compiler_params=pltpu.CompilerParams(use_tc_tiling_on_sc=False),
    )
    return run(kmer_indices, kmer_table)


def kernel(kmer_indices, kmer_table):
    return _encode(kmer_indices, kmer_table)
